# baseline (speedup 1.0000x reference)
import functools

import jax
import jax.numpy as jnp
from jax import lax
from jax.experimental import pallas as pl
from jax.experimental.pallas import tpu as pltpu
from jax.experimental.pallas import tpu_sc as plsc

N = 10000
E = 320000
D = 128
H = 8
DH = 16
HOPS = 4
ALPHA = 0.15
LN_EPS = 1e-5

NC = 2
NS = 16
CH = 128
EPT = E // NS
NCHK = 160
NCHK_R = 157
EPT_P = NCHK * CH
TAIL_REAL = EPT - (NCHK_R - 1) * CH
NP = 10240
RPT = NP // NS
RSUB = 64
NSUB = RPT // RSUB
ZB = 32


def _proj_body(x_ref, wq_ref, wk_ref, wv_ref, q_ref, k_ref, v_ref):
    x = x_ref[...]
    q_ref[...] = jnp.dot(x, wq_ref[0], preferred_element_type=jnp.float32)
    k_ref[...] = jnp.dot(x, wk_ref[0], preferred_element_type=jnp.float32)
    v_ref[...] = jnp.dot(x, wv_ref[0], preferred_element_type=jnp.float32)


_RB = 2048
_NRB = NP // _RB


def _proj(x, wq, wk, wv):
    def split(w):
        return w.reshape(D, NC, 64).transpose(1, 0, 2)
    w_spec = pl.BlockSpec((1, D, 64), lambda i, c: (c, 0, 0))
    o_spec = pl.BlockSpec((_RB, 64), lambda i, c: (c * _NRB + i, 0))
    return pl.pallas_call(
        _proj_body,
        grid=(_NRB, NC),
        in_specs=[
            pl.BlockSpec((_RB, D), lambda i, c: (i, 0)),
            w_spec, w_spec, w_spec,
        ],
        out_specs=[o_spec, o_spec, o_spec],
        out_shape=[jax.ShapeDtypeStruct((NC * NP, 64), jnp.float32)] * 3,
    )(jnp.pad(x, ((0, NP - N), (0, 0))), split(wq), split(wk), split(wv))


def _ln_body(lo_ref, hi_ref, x_ref, g_ref, b_ref, o_ref):
    f = jnp.concatenate([lo_ref[...], hi_ref[...]], axis=1) + x_ref[...]
    mu = jnp.mean(f, axis=1, keepdims=True)
    d = f - mu
    var = jnp.mean(d * d, axis=1, keepdims=True)
    o_ref[...] = d * lax.rsqrt(var + LN_EPS) * g_ref[...] + b_ref[...]


_LB = 2000
_NLB = N // _LB


def _ln(feat2, x, g, b):
    return pl.pallas_call(
        _ln_body,
        grid=(_NLB,),
        in_specs=[
            pl.BlockSpec((_LB, 64), lambda i: (i, 0)),
            pl.BlockSpec((_LB, 64), lambda i: (i, 0)),
            pl.BlockSpec((_LB, D), lambda i: (i, 0)),
            pl.BlockSpec((1, D), lambda i: (0, 0)),
            pl.BlockSpec((1, D), lambda i: (0, 0)),
        ],
        out_specs=pl.BlockSpec((_LB, D), lambda i: (i, 0)),
        out_shape=jax.ShapeDtypeStruct((N, D), jnp.float32),
    )(feat2[:N], feat2[NP:NP + N], x, g.reshape(1, D), b.reshape(1, D))


def _graph_body(q_tab, k_tab, v_tab, src2d, dst2d,
                out_tab,
                esc_hbm, feat_hbm,
                src_sl, dst_sl, qidx, rows, rows2, esc_c, esc_c2,
                esc64, msgb, zbuf, aggv, vv,
                agg_sp,
                sem1, sem2, sg0, sg1, se0, se1, ss0, ss1):
    cid = lax.axis_index("c")
    sid = lax.axis_index("s")
    cid_n = cid * NP
    zero16 = jnp.zeros((16,), jnp.float32)

    pltpu.sync_copy(src2d.at[pl.ds(sid * NCHK, NCHK)], src_sl)
    pltpu.sync_copy(dst2d.at[pl.ds(sid * NCHK, NCHK)], dst_sl)

    @pl.loop(0, NCHK, unroll=2)
    def _off(j):
        @pl.loop(0, CH // 16)
        def _off2(i):
            src_sl[j, pl.ds(i * 16, 16)] = src_sl[j, pl.ds(i * 16, 16)] + cid_n

    @pl.loop(0, CH)
    def _z(r):
        for c4 in range(4):
            esc64[r, pl.ds(c4 * 16, 16)] = zero16

    @pl.loop(0, ZB)
    def _z2(r):
        for c4 in range(4):
            zbuf[r, pl.ds(c4 * 16, 16)] = zero16

    def zero_agg(r0):
        for k in range(RSUB // ZB):
            pltpu.sync_copy(zbuf, agg_sp.at[pl.ds(r0 + k * ZB, ZB)])

    for s in range(NSUB):
        zero_agg(sid * RPT + s * RSUB)
    plsc.subcore_barrier()

    esc_base = (cid * NS + sid) * EPT_P

    @pl.loop(0, NCHK_R)
    def _scores(j):
        @pl.loop(0, CH // 16)
        def _qi(i):
            qidx[pl.ds(i * 16, 16)] = dst_sl[j, pl.ds(i * 16, 16)] + cid_n

        cp1 = pltpu.async_copy(q_tab.at[qidx], rows, sem1)
        cp2 = pltpu.async_copy(k_tab.at[src_sl.at[j]], rows2, sem2)
        cp1.wait()
        cp2.wait()

        lane = lax.iota(jnp.int32, 16)

        def hsum(p):
            for step in (8, 4, 2, 1):
                idx = jnp.bitwise_xor(lane, step)
                p = p + p.at[idx].get(mode="promise_in_bounds")
            return p

        @pl.loop(0, CH, unroll=2)
        def _dot(e):
            acc = jnp.zeros((16,), jnp.float32)
            for h in range(4):
                p = rows[e, pl.ds(h * 16, 16)] * rows2[e, pl.ds(h * 16, 16)]
                acc = jnp.where(lane == h, hsum(p), acc)
            val = jnp.exp(acc * 0.25)
            esc_c[e, :] = val
            esc64[e, pl.ds(0, 16)] = val

        @pl.when(j == NCHK_R - 1)
        def _tail():
            @pl.loop(TAIL_REAL, CH)
            def _tz(e):
                esc_c[e, :] = zero16
                esc64[e, pl.ds(0, 16)] = zero16

        pltpu.sync_copy(esc64, agg_sp.at[dst_sl.at[j]], add=True)
        pltpu.sync_copy(esc_c, esc_hbm.at[pl.ds(esc_base + j * CH, CH)])

    plsc.subcore_barrier()
    for s in range(NSUB):
        r0 = sid * RPT + s * RSUB
        pltpu.sync_copy(agg_sp.at[pl.ds(r0, RSUB)], aggv)

        @pl.loop(0, RSUB, unroll=4)
        def _inv(r):
            aggv[r, pl.ds(0, 16)] = (
                (1.0 - ALPHA) / (aggv[r, pl.ds(0, 16)] + 1e-16))

        pltpu.sync_copy(aggv, agg_sp.at[pl.ds(r0, RSUB)])
    plsc.subcore_barrier()

    RB = (rows, rows2)
    SG = (sg0, sg1)

    def _rs_fire(j, p):
        pltpu.async_copy(agg_sp.at[dst_sl.at[j]], RB[p], SG[p])

    def _rs_wait(p):
        pltpu.make_async_copy(
            agg_sp.at[pl.ds(0, CH)], RB[p], SG[p]).wait()

    def _rs_compute(j, p):
        pltpu.sync_copy(esc_hbm.at[pl.ds(esc_base + j * CH, CH)], esc_c)
        _rs_wait(p)

        @pl.loop(0, CH, unroll=4)
        def _rs(e):
            esc_c[e, :] = esc_c[e, :] * RB[p][e, pl.ds(0, 16)]

        pltpu.sync_copy(esc_c, esc_hbm.at[pl.ds(esc_base + j * CH, CH)])

    _rs_fire(0, 0)
    _rs_fire(1, 1)

    @pl.loop(0, (NCHK_R - 1) // 2)
    def _rescale(k):
        j = 2 * k
        _rs_compute(j, 0)
        _rs_fire(j + 2, 0)
        _rs_compute(j + 1, 1)

        @pl.when(j + 3 < NCHK_R)
        def _():
            _rs_fire(j + 3, 1)

    _rs_compute(NCHK_R - 1, 0)
    plsc.subcore_barrier()

    for s in range(NSUB):
        zero_agg(sid * RPT + s * RSUB)
    plsc.subcore_barrier()

    EB = (esc_c, esc_c2)
    MB = (esc64, msgb)
    SE = (se0, se1)
    SS = (ss0, ss1)

    for s in range(NSUB):
        r0 = sid * RPT + s * RSUB
        pltpu.sync_copy(v_tab.at[pl.ds(cid_n + r0, RSUB)], vv)
        pltpu.sync_copy(vv, feat_hbm.at[pl.ds(cid_n + r0, RSUB)])
    plsc.subcore_barrier()

    def _h_fire(j, p):
        pltpu.async_copy(feat_hbm.at[src_sl.at[j]], RB[p], SG[p])
        pltpu.async_copy(
            esc_hbm.at[pl.ds(esc_base + j * CH, CH)], EB[p], SE[p])

    def _h_wait(p):
        pltpu.make_async_copy(
            feat_hbm.at[pl.ds(0, CH)], RB[p], SG[p]).wait()
        pltpu.make_async_copy(
            esc_hbm.at[pl.ds(0, CH)], EB[p], SE[p]).wait()

    def _h_wait_s(p):
        pltpu.make_async_copy(
            MB[p], agg_sp.at[pl.ds(0, CH)], SS[p]).wait()

    def _h_compute(j, p):
        _h_wait(p)

        @plsc.parallel_loop(0, CH, unroll=8)
        def _scale(e):
            ev = EB[p][e, :]
            for h in range(4):
                MB[p][e, pl.ds(h * 16, 16)] = (
                    ev[h] * RB[p][e, pl.ds(h * 16, 16)])

        pltpu.async_copy(MB[p], agg_sp.at[dst_sl.at[j]], SS[p], add=True)

    @pl.loop(0, HOPS)
    def _hop(t):
        _h_fire(0, 0)
        _h_fire(1, 1)

        @pl.loop(0, (NCHK_R - 1) // 2)
        def _edge(k):
            j = 2 * k

            @pl.when(k > 0)
            def _():
                _h_wait_s(0)

            _h_compute(j, 0)
            _h_fire(j + 2, 0)

            @pl.when(k > 0)
            def _():
                _h_wait_s(1)

            _h_compute(j + 1, 1)

            @pl.when(j + 3 < NCHK_R)
            def _():
                _h_fire(j + 3, 1)

        _h_wait_s(0)
        _h_compute(NCHK_R - 1, 0)
        _h_wait_s(0)
        _h_wait_s(1)
        plsc.subcore_barrier()

        for s in range(NSUB):
            r0 = sid * RPT + s * RSUB
            pltpu.sync_copy(agg_sp.at[pl.ds(r0, RSUB)], aggv)
            pltpu.sync_copy(v_tab.at[pl.ds(cid_n + r0, RSUB)], vv)

            @pl.loop(0, RSUB, unroll=2)
            def _pw(r):
                for h in range(4):
                    aggv[r, pl.ds(h * 16, 16)] = (
                        aggv[r, pl.ds(h * 16, 16)]
                        + ALPHA * vv[r, pl.ds(h * 16, 16)])

            pltpu.sync_copy(aggv, feat_hbm.at[pl.ds(cid_n + r0, RSUB)])

            @pl.when(t == HOPS - 1)
            def _():
                pltpu.sync_copy(aggv, out_tab.at[pl.ds(cid_n + r0, RSUB)])

            zero_agg(r0)
        plsc.subcore_barrier()


def _make_graph():
    mesh = plsc.VectorSubcoreMesh(
        core_axis_name="c", subcore_axis_name="s",
        num_cores=NC, num_subcores=NS)
    return pl.kernel(
        _graph_body,
        out_type=(
            jax.ShapeDtypeStruct((NC * NP, 64), jnp.float32),
            jax.ShapeDtypeStruct((NC * NS * EPT_P, 16), jnp.float32),
            jax.ShapeDtypeStruct((NC * NP, 64), jnp.float32),
        ),
        mesh=mesh,
        scratch_types=[
            pltpu.VMEM((NCHK, CH), jnp.int32),
            pltpu.VMEM((NCHK, CH), jnp.int32),
            pltpu.VMEM((CH,), jnp.int32),
            pltpu.VMEM((CH, 64), jnp.float32),
            pltpu.VMEM((CH, 64), jnp.float32),
            pltpu.VMEM((CH, 16), jnp.float32),
            pltpu.VMEM((CH, 16), jnp.float32),
            pltpu.VMEM((CH, 64), jnp.float32),
            pltpu.VMEM((CH, 64), jnp.float32),
            pltpu.VMEM((ZB, 64), jnp.float32),
            pltpu.VMEM((RSUB, 64), jnp.float32),
            pltpu.VMEM((RSUB, 64), jnp.float32),
            pltpu.VMEM_SHARED((NP, 64), jnp.float32),
        ] + [pltpu.SemaphoreType.DMA] * 8,
        compiler_params=pltpu.CompilerParams(use_tc_tiling_on_sc=False),
    )


def kernel(inputs, edge_index, Wq1, Wk1, Wv1, g1, b1, Wq2, Wk2, Wv2, g2, b2):
    src = edge_index[0]
    dst = edge_index[1]
    padi = jnp.zeros((NS, EPT_P - EPT), jnp.int32)
    src2d = jnp.concatenate([src.reshape(NS, EPT), padi], axis=1)
    src2d = src2d.reshape(NS * NCHK, CH)
    dst2d = jnp.concatenate([dst.reshape(NS, EPT), padi], axis=1)
    dst2d = dst2d.reshape(NS * NCHK, CH)

    graph = _make_graph()

    def layer(x, w):
        wq, wk, wv, g, b = w
        q2, k2, v2 = _proj(x, wq, wk, wv)
        feat2 = graph(q2, k2, v2, src2d, dst2d)[0]
        return _ln(feat2, x, g, b), None

    ws = (jnp.stack([Wq1, Wq2]), jnp.stack([Wk1, Wk2]),
          jnp.stack([Wv1, Wv2]), jnp.stack([g1, g2]), jnp.stack([b1, b2]))
    x, _ = lax.scan(layer, inputs, ws)
    return x


# speedup vs baseline: 1.3335x; 1.3335x over previous
import functools

import jax
import jax.numpy as jnp
from jax import lax
from jax.experimental import pallas as pl
from jax.experimental.pallas import tpu as pltpu
from jax.experimental.pallas import tpu_sc as plsc

N = 10000
E = 320000
D = 128
H = 8
DH = 16
HOPS = 4
ALPHA = 0.15
LN_EPS = 1e-5

NC = 2
NS = 16
CH = 128
EPT = E // NS
NCHK = 160
NCHK_R = 157
EPT_P = NCHK * CH
TAIL_REAL = EPT - (NCHK_R - 1) * CH
NP = 10240
RPT = NP // NS
RSUB = 64
NSUB = RPT // RSUB
ZB = 32


def _proj_body(x_ref, wq_ref, wk_ref, wv_ref, q_ref, k_ref, v_ref):
    x = x_ref[...]
    q_ref[...] = jnp.dot(x, wq_ref[0], preferred_element_type=jnp.float32)
    k_ref[...] = jnp.dot(x, wk_ref[0], preferred_element_type=jnp.float32)
    v_ref[...] = jnp.dot(x, wv_ref[0], preferred_element_type=jnp.float32)


_RB = 2048
_NRB = NP // _RB


def _proj(x, wq, wk, wv):
    def split(w):
        return w.reshape(D, NC, 64).transpose(1, 0, 2)
    w_spec = pl.BlockSpec((1, D, 64), lambda i, c: (c, 0, 0))
    o_spec = pl.BlockSpec((_RB, 64), lambda i, c: (c * _NRB + i, 0))
    return pl.pallas_call(
        _proj_body,
        grid=(_NRB, NC),
        in_specs=[
            pl.BlockSpec((_RB, D), lambda i, c: (i, 0)),
            w_spec, w_spec, w_spec,
        ],
        out_specs=[o_spec, o_spec, o_spec],
        out_shape=[jax.ShapeDtypeStruct((NC * NP, 64), jnp.float32)] * 3,
    )(jnp.pad(x, ((0, NP - N), (0, 0))), split(wq), split(wk), split(wv))


def _ln_body(lo_ref, hi_ref, x_ref, g_ref, b_ref, o_ref):
    f = jnp.concatenate([lo_ref[...], hi_ref[...]], axis=1) + x_ref[...]
    mu = jnp.mean(f, axis=1, keepdims=True)
    d = f - mu
    var = jnp.mean(d * d, axis=1, keepdims=True)
    o_ref[...] = d * lax.rsqrt(var + LN_EPS) * g_ref[...] + b_ref[...]


_LB = 2000
_NLB = N // _LB


def _ln(feat2, x, g, b):
    return pl.pallas_call(
        _ln_body,
        grid=(_NLB,),
        in_specs=[
            pl.BlockSpec((_LB, 64), lambda i: (i, 0)),
            pl.BlockSpec((_LB, 64), lambda i: (i, 0)),
            pl.BlockSpec((_LB, D), lambda i: (i, 0)),
            pl.BlockSpec((1, D), lambda i: (0, 0)),
            pl.BlockSpec((1, D), lambda i: (0, 0)),
        ],
        out_specs=pl.BlockSpec((_LB, D), lambda i: (i, 0)),
        out_shape=jax.ShapeDtypeStruct((N, D), jnp.float32),
    )(feat2[:N], feat2[NP:NP + N], x, g.reshape(1, D), b.reshape(1, D))


def _graph_body(q_tab, k_tab, v_tab, src2d, dst2d,
                out_tab,
                esc_hbm, feat_hbm,
                src_sl, dst_sl, qidx, rows, rows2, esc_c, esc_c2,
                esc64, msgb, zbuf, aggv, vv,
                agg_sp,
                sem1, sem2, sg0, sg1, se0, se1, ss0, ss1):
    cid = lax.axis_index("c")
    sid = lax.axis_index("s")
    cid_n = cid * NP
    zero16 = jnp.zeros((16,), jnp.float32)

    pltpu.sync_copy(src2d.at[pl.ds(sid * NCHK, NCHK)], src_sl)
    pltpu.sync_copy(dst2d.at[pl.ds(sid * NCHK, NCHK)], dst_sl)

    @plsc.parallel_loop(0, NCHK, unroll=2)
    def _off(j):
        @pl.loop(0, CH // 16)
        def _off2(i):
            src_sl[j, pl.ds(i * 16, 16)] = src_sl[j, pl.ds(i * 16, 16)] + cid_n

    @pl.loop(0, CH)
    def _z(r):
        for c4 in range(4):
            esc64[r, pl.ds(c4 * 16, 16)] = zero16

    @pl.loop(0, ZB)
    def _z2(r):
        for c4 in range(4):
            zbuf[r, pl.ds(c4 * 16, 16)] = zero16

    def zero_agg(r0):
        for k in range(RSUB // ZB):
            pltpu.sync_copy(zbuf, agg_sp.at[pl.ds(r0 + k * ZB, ZB)])

    for s in range(NSUB):
        zero_agg(sid * RPT + s * RSUB)
    plsc.subcore_barrier()

    esc_base = (cid * NS + sid) * EPT_P

    @pl.loop(0, NCHK_R)
    def _scores(j):
        @plsc.parallel_loop(0, CH // 16, unroll=4)
        def _qi(i):
            qidx[pl.ds(i * 16, 16)] = dst_sl[j, pl.ds(i * 16, 16)] + cid_n

        cp1 = pltpu.async_copy(q_tab.at[qidx], rows, sem1)
        cp2 = pltpu.async_copy(k_tab.at[src_sl.at[j]], rows2, sem2)
        cp1.wait()
        cp2.wait()

        lane = lax.iota(jnp.int32, 16)

        def hsum(p):
            for step in (8, 4, 2, 1):
                idx = jnp.bitwise_xor(lane, step)
                p = p + p.at[idx].get(mode="promise_in_bounds")
            return p

        @plsc.parallel_loop(0, CH, unroll=4)
        def _dot(e):
            acc = jnp.zeros((16,), jnp.float32)
            for h in range(4):
                p = rows[e, pl.ds(h * 16, 16)] * rows2[e, pl.ds(h * 16, 16)]
                acc = jnp.where(lane == h, hsum(p), acc)
            val = jnp.exp(acc * 0.25)
            esc_c[e, :] = val
            esc64[e, pl.ds(0, 16)] = val

        @pl.when(j == NCHK_R - 1)
        def _tail():
            @pl.loop(TAIL_REAL, CH)
            def _tz(e):
                esc_c[e, :] = zero16
                esc64[e, pl.ds(0, 16)] = zero16

        pltpu.sync_copy(esc64, agg_sp.at[dst_sl.at[j]], add=True)
        pltpu.sync_copy(esc_c, esc_hbm.at[pl.ds(esc_base + j * CH, CH)])

    plsc.subcore_barrier()
    for s in range(NSUB):
        r0 = sid * RPT + s * RSUB
        pltpu.sync_copy(agg_sp.at[pl.ds(r0, RSUB)], aggv)

        @plsc.parallel_loop(0, RSUB, unroll=8)
        def _inv(r):
            aggv[r, pl.ds(0, 16)] = (
                (1.0 - ALPHA) / (aggv[r, pl.ds(0, 16)] + 1e-16))

        pltpu.sync_copy(aggv, agg_sp.at[pl.ds(r0, RSUB)])
    plsc.subcore_barrier()

    RB = (rows, rows2)
    SG = (sg0, sg1)

    def _rs_fire(j, p):
        pltpu.async_copy(agg_sp.at[dst_sl.at[j]], RB[p], SG[p])

    def _rs_wait(p):
        pltpu.make_async_copy(
            agg_sp.at[pl.ds(0, CH)], RB[p], SG[p]).wait()

    def _rs_compute(j, p):
        pltpu.sync_copy(esc_hbm.at[pl.ds(esc_base + j * CH, CH)], esc_c)
        _rs_wait(p)

        @plsc.parallel_loop(0, CH, unroll=8)
        def _rs(e):
            esc_c[e, :] = esc_c[e, :] * RB[p][e, pl.ds(0, 16)]

        pltpu.sync_copy(esc_c, esc_hbm.at[pl.ds(esc_base + j * CH, CH)])

    _rs_fire(0, 0)
    _rs_fire(1, 1)

    @pl.loop(0, (NCHK_R - 1) // 2)
    def _rescale(k):
        j = 2 * k
        _rs_compute(j, 0)
        _rs_fire(j + 2, 0)
        _rs_compute(j + 1, 1)

        @pl.when(j + 3 < NCHK_R)
        def _():
            _rs_fire(j + 3, 1)

    _rs_compute(NCHK_R - 1, 0)
    plsc.subcore_barrier()

    for s in range(NSUB):
        zero_agg(sid * RPT + s * RSUB)
    plsc.subcore_barrier()

    EB = (esc_c, esc_c2)
    MB = (esc64, msgb)
    SE = (se0, se1)
    SS = (ss0, ss1)

    for s in range(NSUB):
        r0 = sid * RPT + s * RSUB
        pltpu.sync_copy(v_tab.at[pl.ds(cid_n + r0, RSUB)], vv)
        pltpu.sync_copy(vv, feat_hbm.at[pl.ds(cid_n + r0, RSUB)])
    plsc.subcore_barrier()

    def _h_fire(j, p):
        pltpu.async_copy(feat_hbm.at[src_sl.at[j]], RB[p], SG[p])
        pltpu.async_copy(
            esc_hbm.at[pl.ds(esc_base + j * CH, CH)], EB[p], SE[p])

    def _h_wait(p):
        pltpu.make_async_copy(
            feat_hbm.at[pl.ds(0, CH)], RB[p], SG[p]).wait()
        pltpu.make_async_copy(
            esc_hbm.at[pl.ds(0, CH)], EB[p], SE[p]).wait()

    def _h_wait_s(p):
        pltpu.make_async_copy(
            MB[p], agg_sp.at[pl.ds(0, CH)], SS[p]).wait()

    def _h_compute(j, p):
        _h_wait(p)

        @plsc.parallel_loop(0, CH, unroll=8)
        def _scale(e):
            ev = EB[p][e, :]
            for h in range(4):
                MB[p][e, pl.ds(h * 16, 16)] = (
                    ev[h] * RB[p][e, pl.ds(h * 16, 16)])

        pltpu.async_copy(MB[p], agg_sp.at[dst_sl.at[j]], SS[p], add=True)

    @pl.loop(0, HOPS)
    def _hop(t):
        _h_fire(0, 0)
        _h_fire(1, 1)

        @pl.loop(0, (NCHK_R - 1) // 2)
        def _edge(k):
            j = 2 * k

            @pl.when(k > 0)
            def _():
                _h_wait_s(0)

            _h_compute(j, 0)
            _h_fire(j + 2, 0)

            @pl.when(k > 0)
            def _():
                _h_wait_s(1)

            _h_compute(j + 1, 1)

            @pl.when(j + 3 < NCHK_R)
            def _():
                _h_fire(j + 3, 1)

        _h_wait_s(0)
        _h_compute(NCHK_R - 1, 0)
        _h_wait_s(0)
        _h_wait_s(1)
        plsc.subcore_barrier()

        for s in range(NSUB):
            r0 = sid * RPT + s * RSUB
            pltpu.sync_copy(agg_sp.at[pl.ds(r0, RSUB)], aggv)
            pltpu.sync_copy(v_tab.at[pl.ds(cid_n + r0, RSUB)], vv)

            @plsc.parallel_loop(0, RSUB, unroll=8)
            def _pw(r):
                for h in range(4):
                    aggv[r, pl.ds(h * 16, 16)] = (
                        aggv[r, pl.ds(h * 16, 16)]
                        + ALPHA * vv[r, pl.ds(h * 16, 16)])

            pltpu.sync_copy(aggv, feat_hbm.at[pl.ds(cid_n + r0, RSUB)])

            @pl.when(t == HOPS - 1)
            def _():
                pltpu.sync_copy(aggv, out_tab.at[pl.ds(cid_n + r0, RSUB)])

            zero_agg(r0)
        plsc.subcore_barrier()


def _make_graph():
    mesh = plsc.VectorSubcoreMesh(
        core_axis_name="c", subcore_axis_name="s",
        num_cores=NC, num_subcores=NS)
    return pl.kernel(
        _graph_body,
        out_type=(
            jax.ShapeDtypeStruct((NC * NP, 64), jnp.float32),
            jax.ShapeDtypeStruct((NC * NS * EPT_P, 16), jnp.float32),
            jax.ShapeDtypeStruct((NC * NP, 64), jnp.float32),
        ),
        mesh=mesh,
        scratch_types=[
            pltpu.VMEM((NCHK, CH), jnp.int32),
            pltpu.VMEM((NCHK, CH), jnp.int32),
            pltpu.VMEM((CH,), jnp.int32),
            pltpu.VMEM((CH, 64), jnp.float32),
            pltpu.VMEM((CH, 64), jnp.float32),
            pltpu.VMEM((CH, 16), jnp.float32),
            pltpu.VMEM((CH, 16), jnp.float32),
            pltpu.VMEM((CH, 64), jnp.float32),
            pltpu.VMEM((CH, 64), jnp.float32),
            pltpu.VMEM((ZB, 64), jnp.float32),
            pltpu.VMEM((RSUB, 64), jnp.float32),
            pltpu.VMEM((RSUB, 64), jnp.float32),
            pltpu.VMEM_SHARED((NP, 64), jnp.float32),
        ] + [pltpu.SemaphoreType.DMA] * 8,
        compiler_params=pltpu.CompilerParams(use_tc_tiling_on_sc=False),
    )


def kernel(inputs, edge_index, Wq1, Wk1, Wv1, g1, b1, Wq2, Wk2, Wv2, g2, b2):
    src = edge_index[0]
    dst = edge_index[1]
    padi = jnp.zeros((NS, EPT_P - EPT), jnp.int32)
    src2d = jnp.concatenate([src.reshape(NS, EPT), padi], axis=1)
    src2d = src2d.reshape(NS * NCHK, CH)
    dst2d = jnp.concatenate([dst.reshape(NS, EPT), padi], axis=1)
    dst2d = dst2d.reshape(NS * NCHK, CH)

    graph = _make_graph()

    def layer(x, w):
        wq, wk, wv, g, b = w
        q2, k2, v2 = _proj(x, wq, wk, wv)
        feat2 = graph(q2, k2, v2, src2d, dst2d)[0]
        return _ln(feat2, x, g, b), None

    ws = (jnp.stack([Wq1, Wq2]), jnp.stack([Wk1, Wk2]),
          jnp.stack([Wv1, Wv2]), jnp.stack([g1, g2]), jnp.stack([b1, b2]))
    x, _ = lax.scan(layer, inputs, ws)
    return x


# speedup vs baseline: 1.3402x; 1.0050x over previous
import jax
import jax.numpy as jnp
from jax import lax
from jax.experimental import pallas as pl
from jax.experimental.pallas import tpu as pltpu
from jax.experimental.pallas import tpu_sc as plsc

N = 10000
E = 320000
D = 128
H = 8
DH = 16
HOPS = 4
ALPHA = 0.15
LN_EPS = 1e-5

NC = 2
NS = 16
CH = 128
EPT = E // NS
NCHK = 160
NCHK_R = 157
EPT_P = NCHK * CH
TAIL_REAL = EPT - (NCHK_R - 1) * CH
NP = 10240
RPT = NP // NS
RSUB = 64
NSUB = RPT // RSUB
ZB = 32


def _proj_body(x_ref, wq_ref, wk_ref, wv_ref, q_ref, k_ref, v_ref):
    x = x_ref[...]
    q_ref[...] = jnp.dot(x, wq_ref[0], preferred_element_type=jnp.float32)
    k_ref[...] = jnp.dot(x, wk_ref[0], preferred_element_type=jnp.float32)
    v_ref[...] = jnp.dot(x, wv_ref[0], preferred_element_type=jnp.float32)


_RB = 2048
_NRB = NP // _RB


def _proj(x, wq, wk, wv):
    def split(w):
        return w.reshape(D, NC, 64).transpose(1, 0, 2)
    w_spec = pl.BlockSpec((1, D, 64), lambda i, c: (c, 0, 0))
    o_spec = pl.BlockSpec((_RB, 64), lambda i, c: (c * _NRB + i, 0))
    return pl.pallas_call(
        _proj_body,
        grid=(_NRB, NC),
        in_specs=[
            pl.BlockSpec((_RB, D), lambda i, c: (i, 0)),
            w_spec, w_spec, w_spec,
        ],
        out_specs=[o_spec, o_spec, o_spec],
        out_shape=[jax.ShapeDtypeStruct((NC * NP, 64), jnp.float32)] * 3,
    )(jnp.pad(x, ((0, NP - N), (0, 0))), split(wq), split(wk), split(wv))


def _ln_body(lo_ref, hi_ref, x_ref, g_ref, b_ref, o_ref):
    f = jnp.concatenate([lo_ref[...], hi_ref[...]], axis=1) + x_ref[...]
    mu = jnp.mean(f, axis=1, keepdims=True)
    d = f - mu
    var = jnp.mean(d * d, axis=1, keepdims=True)
    o_ref[...] = d * lax.rsqrt(var + LN_EPS) * g_ref[...] + b_ref[...]


_LB = 2000
_NLB = N // _LB


def _ln(feat2, x, g, b):
    return pl.pallas_call(
        _ln_body,
        grid=(_NLB,),
        in_specs=[
            pl.BlockSpec((_LB, 64), lambda i: (i, 0)),
            pl.BlockSpec((_LB, 64), lambda i: (i, 0)),
            pl.BlockSpec((_LB, D), lambda i: (i, 0)),
            pl.BlockSpec((1, D), lambda i: (0, 0)),
            pl.BlockSpec((1, D), lambda i: (0, 0)),
        ],
        out_specs=pl.BlockSpec((_LB, D), lambda i: (i, 0)),
        out_shape=jax.ShapeDtypeStruct((N, D), jnp.float32),
    )(feat2[:N], feat2[NP:NP + N], x, g.reshape(1, D), b.reshape(1, D))


def _graph_body(q_tab, k_tab, v_tab, src2d, dst2d,
                out_tab,
                esc_hbm, feat_hbm,
                src_sl, dst_sl, qidx, rows, rows2, esc_c, esc_c2,
                esc64, msgb, zbuf, aggv, vv,
                agg_sp,
                sem1, sem2, sg0, sg1, se0, se1, ss0, ss1):
    cid = lax.axis_index("c")
    sid = lax.axis_index("s")
    cid_n = cid * NP
    zero16 = jnp.zeros((16,), jnp.float32)

    pltpu.sync_copy(src2d.at[pl.ds(sid * NCHK, NCHK)], src_sl)
    pltpu.sync_copy(dst2d.at[pl.ds(sid * NCHK, NCHK)], dst_sl)

    @plsc.parallel_loop(0, NCHK, unroll=2)
    def _off(j):
        @pl.loop(0, CH // 16)
        def _off2(i):
            src_sl[j, pl.ds(i * 16, 16)] = src_sl[j, pl.ds(i * 16, 16)] + cid_n

    @pl.loop(0, CH)
    def _z(r):
        for c4 in range(4):
            esc64[r, pl.ds(c4 * 16, 16)] = zero16

    @pl.loop(0, ZB)
    def _z2(r):
        for c4 in range(4):
            zbuf[r, pl.ds(c4 * 16, 16)] = zero16

    def zero_agg(r0):
        for k in range(RSUB // ZB):
            pltpu.sync_copy(zbuf, agg_sp.at[pl.ds(r0 + k * ZB, ZB)])

    for s in range(NSUB):
        zero_agg(sid * RPT + s * RSUB)
    plsc.subcore_barrier()

    esc_base = (cid * NS + sid) * EPT_P

    @pl.loop(0, NCHK_R)
    def _scores(j):
        @plsc.parallel_loop(0, CH // 16, unroll=4)
        def _qi(i):
            qidx[pl.ds(i * 16, 16)] = dst_sl[j, pl.ds(i * 16, 16)] + cid_n

        cp1 = pltpu.async_copy(q_tab.at[qidx], rows, sem1)
        cp2 = pltpu.async_copy(k_tab.at[src_sl.at[j]], rows2, sem2)
        cp1.wait()
        cp2.wait()

        lane = lax.iota(jnp.int32, 16)

        def hsum(p):
            for step in (8, 4, 2, 1):
                idx = jnp.bitwise_xor(lane, step)
                p = p + p.at[idx].get(mode="promise_in_bounds")
            return p

        @plsc.parallel_loop(0, CH, unroll=4)
        def _dot(e):
            acc = jnp.zeros((16,), jnp.float32)
            for h in range(4):
                p = rows[e, pl.ds(h * 16, 16)] * rows2[e, pl.ds(h * 16, 16)]
                acc = jnp.where(lane == h, hsum(p), acc)
            val = jnp.exp(acc * 0.25)
            esc_c[e, :] = val
            esc64[e, pl.ds(0, 16)] = val

        @pl.when(j == NCHK_R - 1)
        def _tail():
            @pl.loop(TAIL_REAL, CH)
            def _tz(e):
                esc_c[e, :] = zero16
                esc64[e, pl.ds(0, 16)] = zero16

        pltpu.sync_copy(esc64, agg_sp.at[dst_sl.at[j]], add=True)
        pltpu.sync_copy(esc_c, esc_hbm.at[pl.ds(esc_base + j * CH, CH)])

    plsc.subcore_barrier()
    for s in range(NSUB):
        r0 = sid * RPT + s * RSUB
        pltpu.sync_copy(agg_sp.at[pl.ds(r0, RSUB)], aggv)

        @plsc.parallel_loop(0, RSUB, unroll=8)
        def _inv(r):
            aggv[r, pl.ds(0, 16)] = (
                (1.0 - ALPHA) / (aggv[r, pl.ds(0, 16)] + 1e-16))

        pltpu.sync_copy(aggv, agg_sp.at[pl.ds(r0, RSUB)])
    plsc.subcore_barrier()

    RB = (rows, rows2)
    SG = (sg0, sg1)

    def _rs_fire(j, p):
        pltpu.async_copy(agg_sp.at[dst_sl.at[j]], RB[p], SG[p])

    def _rs_wait(p):
        pltpu.make_async_copy(
            agg_sp.at[pl.ds(0, CH)], RB[p], SG[p]).wait()

    def _rs_compute(j, p):
        pltpu.sync_copy(esc_hbm.at[pl.ds(esc_base + j * CH, CH)], esc_c)
        _rs_wait(p)

        @plsc.parallel_loop(0, CH, unroll=8)
        def _rs(e):
            esc_c[e, :] = esc_c[e, :] * RB[p][e, pl.ds(0, 16)]

        pltpu.sync_copy(esc_c, esc_hbm.at[pl.ds(esc_base + j * CH, CH)])

    _rs_fire(0, 0)
    _rs_fire(1, 1)

    @pl.loop(0, (NCHK_R - 1) // 2)
    def _rescale(k):
        j = 2 * k
        _rs_compute(j, 0)
        _rs_fire(j + 2, 0)
        _rs_compute(j + 1, 1)

        @pl.when(j + 3 < NCHK_R)
        def _():
            _rs_fire(j + 3, 1)

    _rs_compute(NCHK_R - 1, 0)
    plsc.subcore_barrier()

    for s in range(NSUB):
        zero_agg(sid * RPT + s * RSUB)
    plsc.subcore_barrier()

    EB = (esc_c, esc_c2)
    MB = (esc64, msgb)
    SE = (se0, se1)
    SS = (ss0, ss1)

    for s in range(NSUB):
        r0 = sid * RPT + s * RSUB
        pltpu.sync_copy(v_tab.at[pl.ds(cid_n + r0, RSUB)], vv)
        pltpu.sync_copy(vv, feat_hbm.at[pl.ds(cid_n + r0, RSUB)])
    plsc.subcore_barrier()

    def _h_fire(j, p):
        pltpu.async_copy(feat_hbm.at[src_sl.at[j]], RB[p], SG[p])
        pltpu.async_copy(
            esc_hbm.at[pl.ds(esc_base + j * CH, CH)], EB[p], SE[p])

    def _h_wait(p):
        pltpu.make_async_copy(
            feat_hbm.at[pl.ds(0, CH)], RB[p], SG[p]).wait()
        pltpu.make_async_copy(
            esc_hbm.at[pl.ds(0, CH)], EB[p], SE[p]).wait()

    def _h_wait_s(p):
        pltpu.make_async_copy(
            MB[p], agg_sp.at[pl.ds(0, CH)], SS[p]).wait()

    def _h_compute(j, p):
        _h_wait(p)

        @plsc.parallel_loop(0, CH, unroll=8)
        def _scale(e):
            ev = EB[p][e, :]
            for h in range(4):
                MB[p][e, pl.ds(h * 16, 16)] = (
                    ev[h] * RB[p][e, pl.ds(h * 16, 16)])

        pltpu.async_copy(MB[p], agg_sp.at[dst_sl.at[j]], SS[p], add=True)

    @pl.loop(0, HOPS)
    def _hop(t):
        _h_fire(0, 0)
        _h_fire(1, 1)

        @pl.loop(0, (NCHK_R - 1) // 2)
        def _edge(k):
            j = 2 * k

            @pl.when(k > 0)
            def _():
                _h_wait_s(0)

            _h_compute(j, 0)
            _h_fire(j + 2, 0)

            @pl.when(k > 0)
            def _():
                _h_wait_s(1)

            _h_compute(j + 1, 1)

            @pl.when(j + 3 < NCHK_R)
            def _():
                _h_fire(j + 3, 1)

        _h_wait_s(0)
        _h_compute(NCHK_R - 1, 0)
        _h_wait_s(0)
        _h_wait_s(1)
        plsc.subcore_barrier()

        for s in range(NSUB):
            r0 = sid * RPT + s * RSUB
            pltpu.sync_copy(agg_sp.at[pl.ds(r0, RSUB)], aggv)
            pltpu.sync_copy(v_tab.at[pl.ds(cid_n + r0, RSUB)], vv)

            @plsc.parallel_loop(0, RSUB, unroll=8)
            def _pw(r):
                for h in range(4):
                    aggv[r, pl.ds(h * 16, 16)] = (
                        aggv[r, pl.ds(h * 16, 16)]
                        + ALPHA * vv[r, pl.ds(h * 16, 16)])

            pltpu.sync_copy(aggv, feat_hbm.at[pl.ds(cid_n + r0, RSUB)])

            @pl.when(t == HOPS - 1)
            def _():
                pltpu.sync_copy(aggv, out_tab.at[pl.ds(cid_n + r0, RSUB)])

            zero_agg(r0)
        plsc.subcore_barrier()


def _make_graph():
    mesh = plsc.VectorSubcoreMesh(
        core_axis_name="c", subcore_axis_name="s",
        num_cores=NC, num_subcores=NS)
    return pl.kernel(
        _graph_body,
        out_type=(
            jax.ShapeDtypeStruct((NC * NP, 64), jnp.float32),
            jax.ShapeDtypeStruct((NC * NS * EPT_P, 16), jnp.float32),
            jax.ShapeDtypeStruct((NC * NP, 64), jnp.float32),
        ),
        mesh=mesh,
        scratch_types=[
            pltpu.VMEM((NCHK, CH), jnp.int32),
            pltpu.VMEM((NCHK, CH), jnp.int32),
            pltpu.VMEM((CH,), jnp.int32),
            pltpu.VMEM((CH, 64), jnp.float32),
            pltpu.VMEM((CH, 64), jnp.float32),
            pltpu.VMEM((CH, 16), jnp.float32),
            pltpu.VMEM((CH, 16), jnp.float32),
            pltpu.VMEM((CH, 64), jnp.float32),
            pltpu.VMEM((CH, 64), jnp.float32),
            pltpu.VMEM((ZB, 64), jnp.float32),
            pltpu.VMEM((RSUB, 64), jnp.float32),
            pltpu.VMEM((RSUB, 64), jnp.float32),
            pltpu.VMEM_SHARED((NP, 64), jnp.float32),
        ] + [pltpu.SemaphoreType.DMA] * 8,
        compiler_params=pltpu.CompilerParams(use_tc_tiling_on_sc=False),
    )


def kernel(inputs, edge_index, Wq1, Wk1, Wv1, g1, b1, Wq2, Wk2, Wv2, g2, b2):
    src = edge_index[0]
    dst = edge_index[1]
    padi = jnp.zeros((NS, EPT_P - EPT), jnp.int32)
    src2d = jnp.concatenate([src.reshape(NS, EPT), padi], axis=1)
    src2d = src2d.reshape(NS * NCHK, CH)
    dst2d = jnp.concatenate([dst.reshape(NS, EPT), padi], axis=1)
    dst2d = dst2d.reshape(NS * NCHK, CH)

    graph = _make_graph()

    def layer(x, w):
        wq, wk, wv, g, b = w
        q2, k2, v2 = _proj(x, wq, wk, wv)
        feat2 = graph(q2, k2, v2, src2d, dst2d)[0]
        return _ln(feat2, x, g, b), None

    ws = (jnp.stack([Wq1, Wq2]), jnp.stack([Wk1, Wk2]),
          jnp.stack([Wv1, Wv2]), jnp.stack([g1, g2]), jnp.stack([b1, b2]))
    x, _ = lax.scan(layer, inputs, ws)
    return x
